# baseline (speedup 1.0000x reference)
import functools

import jax
import jax.numpy as jnp
from jax import lax
from jax.experimental import pallas as pl
from jax.experimental.pallas import tpu as pltpu
from jax.experimental.pallas import tpu_sc as plsc

L = 16
NC = 2
NS = 16
NW = NC * NS
C = 2048


@functools.partial(jax.jit, static_argnums=(4, 5, 6))
def _density_sample(pair_table, xs, ys, zs, D, H, W):
    n = xs.shape[0]
    n_tile = n // NW
    n_chunks = n_tile // C

    sx = (W - 1) * 0.5
    sy = (H - 1) * 0.5
    sz = (D - 1) * 0.5

    mesh = plsc.VectorSubcoreMesh(core_axis_name="c", subcore_axis_name="s")

    @functools.partial(
        pl.kernel,
        out_type=jax.ShapeDtypeStruct((n,), jnp.float32),
        mesh=mesh,
        scratch_types=[
            pltpu.VMEM((C,), jnp.float32),
            pltpu.VMEM((C,), jnp.float32),
            pltpu.VMEM((C,), jnp.float32),
            pltpu.VMEM((C,), jnp.float32),
            pltpu.VMEM((C,), jnp.float32),
            pltpu.VMEM((C,), jnp.float32),
            pltpu.VMEM((4 * C,), jnp.int32),
            pltpu.VMEM((4 * C,), jnp.int32),
            pltpu.VMEM((4 * C,), jnp.int32),
            pltpu.VMEM((4 * C,), jnp.int32),
            pltpu.VMEM((3, C), jnp.float32),
            pltpu.VMEM((3, C), jnp.float32),
            pltpu.VMEM((C,), jnp.float32),
            pltpu.VMEM((C,), jnp.float32),
            pltpu.SemaphoreType.DMA,
            pltpu.SemaphoreType.DMA,
            pltpu.SemaphoreType.DMA,
            pltpu.SemaphoreType.DMA,
            pltpu.SemaphoreType.DMA,
            pltpu.SemaphoreType.DMA,
        ],
        compiler_params=pltpu.CompilerParams(needs_layout_passes=False),
    )
    def body(tab_hbm, xs_hbm, ys_hbm, zs_hbm, out_hbm, x_b0, x_b1, y_b0,
             y_b1, z_b0, z_b1, idx_b0, idx_b1, val_b0, val_b1, w_b0, w_b1,
             out_b0, out_b1, sem_p0, sem_p1, sem_g0, sem_g1, sem_o0, sem_o1):
        x_bufs = (x_b0, x_b1)
        y_bufs = (y_b0, y_b1)
        z_bufs = (z_b0, z_b1)
        idx_bufs = (idx_b0, idx_b1)
        val_bufs = (val_b0, val_b1)
        w_bufs = (w_b0, w_b1)
        out_bufs = (out_b0, out_b1)
        sem_p = (sem_p0, sem_p1)
        sem_g = (sem_g0, sem_g1)
        sem_o = (sem_o0, sem_o1)
        wid = lax.axis_index("s") * NC + lax.axis_index("c")
        tile_base = wid * n_tile

        def start_pts(ci, s):
            b = tile_base + ci * C
            pltpu.async_copy(xs_hbm.at[pl.ds(b, C)], x_bufs[s], sem_p[s])
            pltpu.async_copy(ys_hbm.at[pl.ds(b, C)], y_bufs[s], sem_p[s])
            pltpu.async_copy(zs_hbm.at[pl.ds(b, C)], z_bufs[s], sem_p[s])

        def wait_pts(s):
            pltpu.make_async_copy(xs_hbm.at[pl.ds(0, C)], x_bufs[s], sem_p[s]).wait()
            pltpu.make_async_copy(ys_hbm.at[pl.ds(0, C)], y_bufs[s], sem_p[s]).wait()
            pltpu.make_async_copy(zs_hbm.at[pl.ds(0, C)], z_bufs[s], sem_p[s]).wait()

        def start_gather(s):
            pltpu.async_copy(tab_hbm.at[idx_bufs[s]], val_bufs[s], sem_g[s])

        def wait_gather(s):
            pltpu.make_async_copy(tab_hbm.at[idx_bufs[s]], val_bufs[s], sem_g[s]).wait()

        def start_out(ci, s):
            b = tile_base + ci * C
            pltpu.async_copy(out_bufs[s], out_hbm.at[pl.ds(b, C)], sem_o[s])

        def wait_out(s):
            pltpu.make_async_copy(out_bufs[s], out_hbm.at[pl.ds(0, C)], sem_o[s]).wait()

        def phase1(s):
            x_buf, y_buf, z_buf = x_bufs[s], y_bufs[s], z_bufs[s]
            idx_buf, w_buf = idx_bufs[s], w_bufs[s]

            def group(g, carry):
                b = g * 16
                x = x_buf[pl.ds(b, 16)]
                y = y_buf[pl.ds(b, 16)]
                z = z_buf[pl.ds(b, 16)]
                ix = jnp.clip((x + 1.0) * sx, 0.0, W - 1)
                iy = jnp.clip((y + 1.0) * sy, 0.0, H - 1)
                iz = jnp.clip((z + 1.0) * sz, 0.0, D - 1)
                x0 = ix.astype(jnp.int32)
                y0 = iy.astype(jnp.int32)
                z0 = iz.astype(jnp.int32)
                w_buf[0, pl.ds(b, 16)] = ix - x0.astype(jnp.float32)
                w_buf[1, pl.ds(b, 16)] = iy - y0.astype(jnp.float32)
                w_buf[2, pl.ds(b, 16)] = iz - z0.astype(jnp.float32)
                y1 = jnp.minimum(y0 + 1, H - 1)
                z1 = jnp.minimum(z0 + 1, D - 1)
                xp = ((x0 >> 7) << 10) + (x0 & 127)
                yp0 = ((y0 >> 3) << 11) + ((y0 & 7) << 7)
                yp1 = ((y1 >> 3) << 11) + ((y1 & 7) << 7)
                zp0 = z0 << 16
                zp1 = z1 << 16
                i00 = zp0 + yp0 + xp
                i01 = zp0 + yp1 + xp
                i10 = zp1 + yp0 + xp
                i11 = zp1 + yp1 + xp
                idx_buf[pl.ds(0 * C + b, 16)] = i00
                idx_buf[pl.ds(1 * C + b, 16)] = i01
                idx_buf[pl.ds(2 * C + b, 16)] = i10
                idx_buf[pl.ds(3 * C + b, 16)] = i11
                return carry

            lax.fori_loop(0, C // 16, group, 0, unroll=False)

        def phase2(s):
            val_buf, w_buf, out_buf = val_bufs[s], w_bufs[s], out_bufs[s]
            himask = jnp.int32(-65536)

            def group(g, carry):
                b = g * 16
                w00 = val_buf[pl.ds(0 * C + b, 16)]
                w01 = val_buf[pl.ds(1 * C + b, 16)]
                w10 = val_buf[pl.ds(2 * C + b, 16)]
                w11 = val_buf[pl.ds(3 * C + b, 16)]
                wx = w_buf[0, pl.ds(b, 16)]
                wy = w_buf[1, pl.ds(b, 16)]
                wz = w_buf[2, pl.ds(b, 16)]
                c000 = plsc.bitcast(w00 << 16, jnp.float32)
                c001 = plsc.bitcast(w00 & himask, jnp.float32)
                c010 = plsc.bitcast(w01 << 16, jnp.float32)
                c011 = plsc.bitcast(w01 & himask, jnp.float32)
                c100 = plsc.bitcast(w10 << 16, jnp.float32)
                c101 = plsc.bitcast(w10 & himask, jnp.float32)
                c110 = plsc.bitcast(w11 << 16, jnp.float32)
                c111 = plsc.bitcast(w11 & himask, jnp.float32)
                a = c000 + wx * (c001 - c000)
                bq = c010 + wx * (c011 - c010)
                e = c100 + wx * (c101 - c100)
                f = c110 + wx * (c111 - c110)
                ab = a + wy * (bq - a)
                ef = e + wy * (f - e)
                out_buf[pl.ds(b, 16)] = ab + wz * (ef - ab)
                return carry

            lax.fori_loop(0, C // 16, group, 0, unroll=False)

        for s in (0, 1):
            start_pts(s, s)
        for s in (0, 1):
            wait_pts(s)
            phase1(s)
            start_gather(s)

        def loop_body(ci2, carry):
            for s in (0, 1):
                ci = ci2 * 2 + s
                nxt = ci + 2

                @pl.when(nxt < n_chunks)
                def _():
                    start_pts(nxt, s)

                wait_gather(s)

                @pl.when(ci >= 2)
                def _():
                    wait_out(s)

                phase2(s)
                start_out(ci, s)

                @pl.when(nxt < n_chunks)
                def _():
                    wait_pts(s)
                    phase1(s)
                    start_gather(s)
            return carry

        lax.fori_loop(0, n_chunks // 2, loop_body, 0, unroll=False)
        for s in (0, 1):
            wait_out(s)

    return body(pair_table, xs, ys, zs)


BLK = 1 << 19
TAIL = 1024


def _build_table(vol_i32):
    n = vol_i32.shape[0]
    grid = n // BLK

    def kern(v_hbm, t_hbm, vbuf, obuf, sem_i, sem_o):
        g = pl.program_id(0)
        base = g * BLK
        pltpu.async_copy(v_hbm.at[pl.ds(base, BLK)], vbuf.at[pl.ds(0, BLK)],
                         sem_i).wait()
        tb = jnp.minimum(base + BLK, n - TAIL)
        pltpu.async_copy(v_hbm.at[pl.ds(tb, TAIL)],
                         vbuf.at[pl.ds(BLK, TAIL)], sem_i).wait()

        def rnd(v):
            return (v + ((v >> 16) & 1) + 0x7FFF) >> 16

        a = rnd(vbuf[pl.ds(0, BLK)])
        b = rnd(vbuf[pl.ds(1, BLK)])
        c = rnd(vbuf[pl.ds(897, BLK)])
        j = lax.iota(jnp.int32, BLK) + base
        hi = jnp.where((j & 1151) == 127, c, b)
        obuf[...] = a | (hi << 16)
        pltpu.async_copy(obuf, t_hbm.at[pl.ds(base, BLK)], sem_o).wait()

    return pl.pallas_call(
        kern,
        grid=(grid,),
        in_specs=[pl.BlockSpec(memory_space=pl.ANY)],
        out_specs=pl.BlockSpec(memory_space=pl.ANY),
        out_shape=jax.ShapeDtypeStruct((n,), jnp.int32),
        scratch_shapes=[
            pltpu.VMEM((BLK + TAIL,), jnp.int32),
            pltpu.VMEM((BLK,), jnp.int32),
            pltpu.SemaphoreType.DMA,
            pltpu.SemaphoreType.DMA,
        ],
    )(vol_i32)


def kernel(density_volume, pts):
    _, D, H, W = density_volume.shape
    vol_tiles = (
        density_volume.reshape(D, H // 8, 8, W // 128, 128)
        .transpose(0, 1, 3, 2, 4)
        .reshape(-1)
    )
    vol_i32 = jax.lax.bitcast_convert_type(vol_tiles, jnp.int32)
    pair_table = _build_table(vol_i32)
    return _density_sample(pair_table, pts[:, 0], pts[:, 1], pts[:, 2], D, H, W)


# speedup vs baseline: 1.2021x; 1.2021x over previous
import functools

import jax
import jax.numpy as jnp
from jax import lax
from jax.experimental import pallas as pl
from jax.experimental.pallas import tpu as pltpu
from jax.experimental.pallas import tpu_sc as plsc

L = 16
NC = 2
NS = 16
NW = NC * NS
C = 2048


@functools.partial(jax.jit, static_argnums=(4, 5, 6))
def _density_sample(pair_table, xs, ys, zs, D, H, W):
    n = xs.shape[0]
    n_tile = n // NW
    n_chunks = n_tile // C

    sx = (W - 1) * 0.5
    sy = (H - 1) * 0.5
    sz = (D - 1) * 0.5

    mesh = plsc.VectorSubcoreMesh(core_axis_name="c", subcore_axis_name="s")

    @functools.partial(
        pl.kernel,
        out_type=jax.ShapeDtypeStruct((n,), jnp.float32),
        mesh=mesh,
        scratch_types=[
            pltpu.VMEM((C,), jnp.float32),
            pltpu.VMEM((C,), jnp.float32),
            pltpu.VMEM((C,), jnp.float32),
            pltpu.VMEM((C,), jnp.float32),
            pltpu.VMEM((C,), jnp.float32),
            pltpu.VMEM((C,), jnp.float32),
            pltpu.VMEM((4 * C,), jnp.int32),
            pltpu.VMEM((4 * C,), jnp.int32),
            pltpu.VMEM((4 * C,), jnp.int32),
            pltpu.VMEM((4 * C,), jnp.int32),
            pltpu.VMEM((3, C), jnp.float32),
            pltpu.VMEM((3, C), jnp.float32),
            pltpu.VMEM((C,), jnp.float32),
            pltpu.VMEM((C,), jnp.float32),
            pltpu.SemaphoreType.DMA,
            pltpu.SemaphoreType.DMA,
            pltpu.SemaphoreType.DMA,
            pltpu.SemaphoreType.DMA,
            pltpu.SemaphoreType.DMA,
            pltpu.SemaphoreType.DMA,
        ],
        compiler_params=pltpu.CompilerParams(needs_layout_passes=False),
    )
    def body(tab_hbm, xs_hbm, ys_hbm, zs_hbm, out_hbm, x_b0, x_b1, y_b0,
             y_b1, z_b0, z_b1, idx_b0, idx_b1, val_b0, val_b1, w_b0, w_b1,
             out_b0, out_b1, sem_p0, sem_p1, sem_g0, sem_g1, sem_o0, sem_o1):
        x_bufs = (x_b0, x_b1)
        y_bufs = (y_b0, y_b1)
        z_bufs = (z_b0, z_b1)
        idx_bufs = (idx_b0, idx_b1)
        val_bufs = (val_b0, val_b1)
        w_bufs = (w_b0, w_b1)
        out_bufs = (out_b0, out_b1)
        sem_p = (sem_p0, sem_p1)
        sem_g = (sem_g0, sem_g1)
        sem_o = (sem_o0, sem_o1)
        wid = lax.axis_index("s") * NC + lax.axis_index("c")
        tile_base = wid * n_tile

        def start_pts(ci, s):
            b = tile_base + ci * C
            pltpu.async_copy(xs_hbm.at[pl.ds(b, C)], x_bufs[s], sem_p[s])
            pltpu.async_copy(ys_hbm.at[pl.ds(b, C)], y_bufs[s], sem_p[s])
            pltpu.async_copy(zs_hbm.at[pl.ds(b, C)], z_bufs[s], sem_p[s])

        def wait_pts(s):
            pltpu.make_async_copy(xs_hbm.at[pl.ds(0, C)], x_bufs[s], sem_p[s]).wait()
            pltpu.make_async_copy(ys_hbm.at[pl.ds(0, C)], y_bufs[s], sem_p[s]).wait()
            pltpu.make_async_copy(zs_hbm.at[pl.ds(0, C)], z_bufs[s], sem_p[s]).wait()

        def start_gather(s):
            pltpu.async_copy(tab_hbm.at[idx_bufs[s]], val_bufs[s], sem_g[s])

        def wait_gather(s):
            pltpu.make_async_copy(tab_hbm.at[idx_bufs[s]], val_bufs[s], sem_g[s]).wait()

        def start_out(ci, s):
            b = tile_base + ci * C
            pltpu.async_copy(out_bufs[s], out_hbm.at[pl.ds(b, C)], sem_o[s])

        def wait_out(s):
            pltpu.make_async_copy(out_bufs[s], out_hbm.at[pl.ds(0, C)], sem_o[s]).wait()

        def phase1(s):
            x_buf, y_buf, z_buf = x_bufs[s], y_bufs[s], z_bufs[s]
            idx_buf, w_buf = idx_bufs[s], w_bufs[s]

            def group(g, carry):
                b = g * 16
                x = x_buf[pl.ds(b, 16)]
                y = y_buf[pl.ds(b, 16)]
                z = z_buf[pl.ds(b, 16)]
                ix = jnp.clip((x + 1.0) * sx, 0.0, W - 1)
                iy = jnp.clip((y + 1.0) * sy, 0.0, H - 1)
                iz = jnp.clip((z + 1.0) * sz, 0.0, D - 1)
                x0 = ix.astype(jnp.int32)
                y0 = iy.astype(jnp.int32)
                z0 = iz.astype(jnp.int32)
                w_buf[0, pl.ds(b, 16)] = ix - x0.astype(jnp.float32)
                w_buf[1, pl.ds(b, 16)] = iy - y0.astype(jnp.float32)
                w_buf[2, pl.ds(b, 16)] = iz - z0.astype(jnp.float32)
                y1 = jnp.minimum(y0 + 1, H - 1)
                z1 = jnp.minimum(z0 + 1, D - 1)
                xp = ((x0 >> 7) << 10) + (x0 & 127)
                yp0 = ((y0 >> 3) << 11) + ((y0 & 7) << 7)
                yp1 = ((y1 >> 3) << 11) + ((y1 & 7) << 7)
                zp0 = z0 << 16
                zp1 = z1 << 16
                i00 = zp0 + yp0 + xp
                i01 = zp0 + yp1 + xp
                i10 = zp1 + yp0 + xp
                i11 = zp1 + yp1 + xp
                idx_buf[pl.ds(0 * C + b, 16)] = i00
                idx_buf[pl.ds(1 * C + b, 16)] = i01
                idx_buf[pl.ds(2 * C + b, 16)] = i10
                idx_buf[pl.ds(3 * C + b, 16)] = i11
                return carry

            lax.fori_loop(0, C // 16, group, 0, unroll=False)

        def phase2(s):
            val_buf, w_buf, out_buf = val_bufs[s], w_bufs[s], out_bufs[s]
            himask = jnp.int32(-65536)

            def group(g, carry):
                b = g * 16
                w00 = val_buf[pl.ds(0 * C + b, 16)]
                w01 = val_buf[pl.ds(1 * C + b, 16)]
                w10 = val_buf[pl.ds(2 * C + b, 16)]
                w11 = val_buf[pl.ds(3 * C + b, 16)]
                wx = w_buf[0, pl.ds(b, 16)]
                wy = w_buf[1, pl.ds(b, 16)]
                wz = w_buf[2, pl.ds(b, 16)]
                c000 = plsc.bitcast(w00 << 16, jnp.float32)
                c001 = plsc.bitcast(w00 & himask, jnp.float32)
                c010 = plsc.bitcast(w01 << 16, jnp.float32)
                c011 = plsc.bitcast(w01 & himask, jnp.float32)
                c100 = plsc.bitcast(w10 << 16, jnp.float32)
                c101 = plsc.bitcast(w10 & himask, jnp.float32)
                c110 = plsc.bitcast(w11 << 16, jnp.float32)
                c111 = plsc.bitcast(w11 & himask, jnp.float32)
                a = c000 + wx * (c001 - c000)
                bq = c010 + wx * (c011 - c010)
                e = c100 + wx * (c101 - c100)
                f = c110 + wx * (c111 - c110)
                ab = a + wy * (bq - a)
                ef = e + wy * (f - e)
                out_buf[pl.ds(b, 16)] = ab + wz * (ef - ab)
                return carry

            lax.fori_loop(0, C // 16, group, 0, unroll=False)

        for s in (0, 1):
            start_pts(s, s)
        for s in (0, 1):
            wait_pts(s)
            phase1(s)
            start_gather(s)

        def loop_body(ci2, carry):
            for s in (0, 1):
                ci = ci2 * 2 + s
                nxt = ci + 2

                @pl.when(nxt < n_chunks)
                def _():
                    start_pts(nxt, s)

                wait_gather(s)

                @pl.when(ci >= 2)
                def _():
                    wait_out(s)

                phase2(s)
                start_out(ci, s)

                @pl.when(nxt < n_chunks)
                def _():
                    wait_pts(s)
                    phase1(s)
                    start_gather(s)
            return carry

        lax.fori_loop(0, n_chunks // 2, loop_body, 0, unroll=False)
        for s in (0, 1):
            wait_out(s)

    return body(pair_table, xs, ys, zs)


BLK = 1 << 19


def _build_table(vol_i32):
    n = vol_i32.shape[0]
    grid = n // BLK

    def kern(vblk, tblk):
        raw = vblk[...]
        a = (raw + ((raw >> 16) & 1) + 0x7FFF) >> 16
        b = jnp.roll(a, -1)
        c = jnp.roll(a, -897)
        j = lax.iota(jnp.int32, BLK) + pl.program_id(0) * BLK
        hi = jnp.where((j & 1151) == 127, c, b)
        hi = jnp.where((j & 2047) == 2047, a, hi)
        tblk[...] = a | (hi << 16)

    return pl.pallas_call(
        kern,
        grid=(grid,),
        in_specs=[pl.BlockSpec((BLK,), lambda i: (i,))],
        out_specs=pl.BlockSpec((BLK,), lambda i: (i,)),
        out_shape=jax.ShapeDtypeStruct((n,), jnp.int32),
    )(vol_i32)


def kernel(density_volume, pts):
    _, D, H, W = density_volume.shape
    vol_tiles = (
        density_volume.reshape(D, H // 8, 8, W // 128, 128)
        .transpose(0, 1, 3, 2, 4)
        .reshape(-1)
    )
    vol_i32 = jax.lax.bitcast_convert_type(vol_tiles, jnp.int32)
    pair_table = _build_table(vol_i32)
    return _density_sample(pair_table, pts[:, 0], pts[:, 1], pts[:, 2], D, H, W)


# speedup vs baseline: 1.9317x; 1.6069x over previous
import functools

import jax
import jax.numpy as jnp
from jax import lax
from jax.experimental import pallas as pl
from jax.experimental.pallas import tpu as pltpu
from jax.experimental.pallas import tpu_sc as plsc

L = 16
NC = 2
NS = 16
NW = NC * NS
C = 2048


@functools.partial(jax.jit, static_argnums=(4, 5, 6))
def _density_sample(vol_tiles, xs, ys, zs, D, H, W):
    n = xs.shape[0]
    n_tile = n // NW
    n_chunks = n_tile // C

    sx = (W - 1) * 0.5
    sy = (H - 1) * 0.5
    sz = (D - 1) * 0.5

    mesh = plsc.VectorSubcoreMesh(core_axis_name="c", subcore_axis_name="s")

    @functools.partial(
        pl.kernel,
        out_type=jax.ShapeDtypeStruct((n,), jnp.float32),
        mesh=mesh,
        scratch_types=[
            pltpu.VMEM((C,), jnp.float32),
            pltpu.VMEM((C,), jnp.float32),
            pltpu.VMEM((C,), jnp.float32),
            pltpu.VMEM((C,), jnp.float32),
            pltpu.VMEM((C,), jnp.float32),
            pltpu.VMEM((C,), jnp.float32),
            pltpu.VMEM((8 * C,), jnp.int32),
            pltpu.VMEM((8 * C,), jnp.int32),
            pltpu.VMEM((8 * C,), jnp.float32),
            pltpu.VMEM((8 * C,), jnp.float32),
            pltpu.VMEM((3, C), jnp.float32),
            pltpu.VMEM((3, C), jnp.float32),
            pltpu.VMEM((C,), jnp.float32),
            pltpu.VMEM((C,), jnp.float32),
            pltpu.SemaphoreType.DMA,
            pltpu.SemaphoreType.DMA,
            pltpu.SemaphoreType.DMA,
            pltpu.SemaphoreType.DMA,
            pltpu.SemaphoreType.DMA,
            pltpu.SemaphoreType.DMA,
        ],
        compiler_params=pltpu.CompilerParams(needs_layout_passes=False),
    )
    def body(vol_hbm, xs_hbm, ys_hbm, zs_hbm, out_hbm, x_b0, x_b1, y_b0,
             y_b1, z_b0, z_b1, idx_b0, idx_b1, val_b0, val_b1, w_b0, w_b1,
             out_b0, out_b1, sem_p0, sem_p1, sem_g0, sem_g1, sem_o0, sem_o1):
        x_bufs = (x_b0, x_b1)
        y_bufs = (y_b0, y_b1)
        z_bufs = (z_b0, z_b1)
        idx_bufs = (idx_b0, idx_b1)
        val_bufs = (val_b0, val_b1)
        w_bufs = (w_b0, w_b1)
        out_bufs = (out_b0, out_b1)
        sem_p = (sem_p0, sem_p1)
        sem_g = (sem_g0, sem_g1)
        sem_o = (sem_o0, sem_o1)
        wid = lax.axis_index("s") * NC + lax.axis_index("c")
        tile_base = wid * n_tile

        def start_pts(ci, s):
            b = tile_base + ci * C
            pltpu.async_copy(xs_hbm.at[pl.ds(b, C)], x_bufs[s], sem_p[s])
            pltpu.async_copy(ys_hbm.at[pl.ds(b, C)], y_bufs[s], sem_p[s])
            pltpu.async_copy(zs_hbm.at[pl.ds(b, C)], z_bufs[s], sem_p[s])

        def wait_pts(s):
            pltpu.make_async_copy(xs_hbm.at[pl.ds(0, C)], x_bufs[s], sem_p[s]).wait()
            pltpu.make_async_copy(ys_hbm.at[pl.ds(0, C)], y_bufs[s], sem_p[s]).wait()
            pltpu.make_async_copy(zs_hbm.at[pl.ds(0, C)], z_bufs[s], sem_p[s]).wait()

        def start_gather(s):
            pltpu.async_copy(vol_hbm.at[idx_bufs[s]], val_bufs[s], sem_g[s])

        def wait_gather(s):
            pltpu.make_async_copy(vol_hbm.at[idx_bufs[s]], val_bufs[s], sem_g[s]).wait()

        def start_out(ci, s):
            b = tile_base + ci * C
            pltpu.async_copy(out_bufs[s], out_hbm.at[pl.ds(b, C)], sem_o[s])

        def wait_out(s):
            pltpu.make_async_copy(out_bufs[s], out_hbm.at[pl.ds(0, C)], sem_o[s]).wait()

        def phase1(s):
            x_buf, y_buf, z_buf = x_bufs[s], y_bufs[s], z_bufs[s]
            idx_buf, w_buf = idx_bufs[s], w_bufs[s]

            def group(g, carry):
                b = g * 16
                x = x_buf[pl.ds(b, 16)]
                y = y_buf[pl.ds(b, 16)]
                z = z_buf[pl.ds(b, 16)]
                ix = jnp.clip((x + 1.0) * sx, 0.0, W - 1)
                iy = jnp.clip((y + 1.0) * sy, 0.0, H - 1)
                iz = jnp.clip((z + 1.0) * sz, 0.0, D - 1)
                x0 = ix.astype(jnp.int32)
                y0 = iy.astype(jnp.int32)
                z0 = iz.astype(jnp.int32)
                w_buf[0, pl.ds(b, 16)] = ix - x0.astype(jnp.float32)
                w_buf[1, pl.ds(b, 16)] = iy - y0.astype(jnp.float32)
                w_buf[2, pl.ds(b, 16)] = iz - z0.astype(jnp.float32)
                x1 = jnp.minimum(x0 + 1, W - 1)
                y1 = jnp.minimum(y0 + 1, H - 1)
                z1 = jnp.minimum(z0 + 1, D - 1)
                xp0 = ((x0 >> 7) << 10) + (x0 & 127)
                xp1 = ((x1 >> 7) << 10) + (x1 & 127)
                yp0 = ((y0 >> 3) << 11) + ((y0 & 7) << 7)
                yp1 = ((y1 >> 3) << 11) + ((y1 & 7) << 7)
                a00 = (z0 << 16) + yp0
                a01 = (z0 << 16) + yp1
                a10 = (z1 << 16) + yp0
                a11 = (z1 << 16) + yp1
                idx_buf[pl.ds(0 * C + b, 16)] = a00 + xp0
                idx_buf[pl.ds(1 * C + b, 16)] = a00 + xp1
                idx_buf[pl.ds(2 * C + b, 16)] = a01 + xp0
                idx_buf[pl.ds(3 * C + b, 16)] = a01 + xp1
                idx_buf[pl.ds(4 * C + b, 16)] = a10 + xp0
                idx_buf[pl.ds(5 * C + b, 16)] = a10 + xp1
                idx_buf[pl.ds(6 * C + b, 16)] = a11 + xp0
                idx_buf[pl.ds(7 * C + b, 16)] = a11 + xp1
                return carry

            lax.fori_loop(0, C // 16, group, 0, unroll=False)

        def phase2(s):
            val_buf, w_buf, out_buf = val_bufs[s], w_bufs[s], out_bufs[s]

            def group(g, carry):
                b = g * 16
                c000 = val_buf[pl.ds(0 * C + b, 16)]
                c001 = val_buf[pl.ds(1 * C + b, 16)]
                c010 = val_buf[pl.ds(2 * C + b, 16)]
                c011 = val_buf[pl.ds(3 * C + b, 16)]
                c100 = val_buf[pl.ds(4 * C + b, 16)]
                c101 = val_buf[pl.ds(5 * C + b, 16)]
                c110 = val_buf[pl.ds(6 * C + b, 16)]
                c111 = val_buf[pl.ds(7 * C + b, 16)]
                wx = w_buf[0, pl.ds(b, 16)]
                wy = w_buf[1, pl.ds(b, 16)]
                wz = w_buf[2, pl.ds(b, 16)]
                a = c000 + wx * (c001 - c000)
                bq = c010 + wx * (c011 - c010)
                e = c100 + wx * (c101 - c100)
                f = c110 + wx * (c111 - c110)
                ab = a + wy * (bq - a)
                ef = e + wy * (f - e)
                out_buf[pl.ds(b, 16)] = ab + wz * (ef - ab)
                return carry

            lax.fori_loop(0, C // 16, group, 0, unroll=False)

        for s in (0, 1):
            start_pts(s, s)
        for s in (0, 1):
            wait_pts(s)
            phase1(s)
            start_gather(s)

        def loop_body(ci2, carry):
            for s in (0, 1):
                ci = ci2 * 2 + s
                nxt = ci + 2

                @pl.when(nxt < n_chunks)
                def _():
                    start_pts(nxt, s)

                wait_gather(s)

                @pl.when(ci >= 2)
                def _():
                    wait_out(s)

                phase2(s)
                start_out(ci, s)

                @pl.when(nxt < n_chunks)
                def _():
                    wait_pts(s)
                    phase1(s)
                    start_gather(s)
            return carry

        lax.fori_loop(0, n_chunks // 2, loop_body, 0, unroll=False)
        for s in (0, 1):
            wait_out(s)

    return body(vol_tiles, xs, ys, zs)


def kernel(density_volume, pts):
    _, D, H, W = density_volume.shape
    vol_tiles = (
        density_volume.reshape(D, H // 8, 8, W // 128, 128)
        .transpose(0, 1, 3, 2, 4)
        .reshape(-1)
    )
    return _density_sample(vol_tiles, pts[:, 0], pts[:, 1], pts[:, 2], D, H, W)
